# baseline (speedup 1.0000x reference)
import functools
import math

import jax
import jax.numpy as jnp
from jax import lax
from jax.experimental import pallas as pl
from jax.experimental.pallas import tpu as pltpu
from jax.experimental.pallas import tpu_sc as plsc

D = 128
SEQ = 200
BATCH = 4096
VOCAB = 4
NC, NS = 2, 16
NW = NC * NS
ROWS = BATCH * SEQ
RPW = ROWS // NW
CHUNK = 128
NCHUNK = RPW // CHUNK
GROUP = 2
NSTEP = NCHUNK // GROUP
NPAIR = NSTEP // 2
TROWS = SEQ * VOCAB


PREP_GRID = 8
PREP_BLK = BATCH // PREP_GRID


def _prep_body(tok_ref, table_ref, pe_ref, comb_ref, gidx_ref):
    scale = jnp.float32(math.sqrt(float(D)))

    @pl.when(pl.program_id(0) == 0)
    def _comb():
        pe = pe_ref[...]
        for v in range(VOCAB):
            comb_ref[:, v, :] = pe + table_ref[v, :][None, :] * scale

    pos = lax.broadcasted_iota(jnp.int32, (PREP_BLK, SEQ), 1)
    gidx_ref[...] = tok_ref[...] + VOCAB * pos


def _sc_body(comb_hbm, gidx_hbm, out_hbm, idx_v, buf_a, buf_b, comb_sh,
             gsem_a, gsem_b, ssem_a, ssem_b):
    cid = lax.axis_index("c")
    sid = lax.axis_index("s")
    wid = sid * NC + cid
    base = wid * RPW

    srows = 80

    @pl.when(sid < TROWS // srows)
    def _stage():
        off = pl.multiple_of(sid * srows, 8)
        pltpu.sync_copy(comb_hbm.at[pl.ds(off, srows)], buf_a.at[pl.ds(0, srows)])
        pltpu.sync_copy(buf_a.at[pl.ds(0, srows)], comb_sh.at[pl.ds(off, srows)])

    plsc.subcore_barrier()

    pltpu.sync_copy(gidx_hbm.at[wid], idx_v)

    def gather(s, buf, sem, issue):
        for g in range(GROUP):
            cp = pltpu.make_async_copy(
                comb_sh.at[idx_v.at[s * GROUP + g]],
                buf.at[pl.ds(g * CHUNK, CHUNK)],
                sem,
            )
            if issue:
                cp.start()
            else:
                cp.wait()

    def scatter(s, buf, sem, issue):
        cp = pltpu.make_async_copy(
            buf, out_hbm.at[pl.ds(base + s * GROUP * CHUNK, GROUP * CHUNK)], sem
        )
        if issue:
            cp.start()
        else:
            cp.wait()

    gather(0, buf_a, gsem_a, True)

    def body(k, carry):
        s0 = 2 * k

        @pl.when(k > 0)
        def _():
            scatter(s0 - 1, buf_b, ssem_b, False)

        gather(s0 + 1, buf_b, gsem_b, True)
        gather(s0, buf_a, gsem_a, False)
        scatter(s0, buf_a, ssem_a, True)

        scatter(s0, buf_a, ssem_a, False)

        @pl.when(k < NPAIR - 1)
        def _():
            gather(s0 + 2, buf_a, gsem_a, True)

        gather(s0 + 1, buf_b, gsem_b, False)
        scatter(s0 + 1, buf_b, ssem_b, True)
        return carry

    lax.fori_loop(0, NPAIR, body, 0)
    scatter(NSTEP - 1, buf_b, ssem_b, False)


def kernel(token_indices, table, pe):
    comb, gidx = pl.pallas_call(
        _prep_body,
        grid=(PREP_GRID,),
        in_specs=[
            pl.BlockSpec((PREP_BLK, SEQ), lambda i: (i, 0)),
            pl.BlockSpec((VOCAB, D), lambda i: (0, 0)),
            pl.BlockSpec((SEQ, D), lambda i: (0, 0)),
        ],
        out_specs=[
            pl.BlockSpec((SEQ, VOCAB, D), lambda i: (0, 0, 0)),
            pl.BlockSpec((PREP_BLK, SEQ), lambda i: (i, 0)),
        ],
        out_shape=(
            jax.ShapeDtypeStruct((SEQ, VOCAB, D), jnp.float32),
            jax.ShapeDtypeStruct((BATCH, SEQ), jnp.int32),
        ),
    )(token_indices, table, pe[:SEQ])

    comb = comb.reshape(TROWS, D)
    gidx3 = gidx.reshape(NW, NCHUNK, CHUNK)

    sc = pl.kernel(
        _sc_body,
        out_type=jax.ShapeDtypeStruct((ROWS, D), jnp.float32),
        mesh=plsc.VectorSubcoreMesh(
            core_axis_name="c", subcore_axis_name="s", num_cores=NC, num_subcores=NS
        ),
        scratch_types=[
            pltpu.VMEM((NCHUNK, CHUNK), jnp.int32),
            pltpu.VMEM((GROUP * CHUNK, D), jnp.float32),
            pltpu.VMEM((GROUP * CHUNK, D), jnp.float32),
            pltpu.VMEM_SHARED((TROWS, D), jnp.float32),
            pltpu.SemaphoreType.DMA,
            pltpu.SemaphoreType.DMA,
            pltpu.SemaphoreType.DMA,
            pltpu.SemaphoreType.DMA,
        ],
    )
    out = sc(comb, gidx3)
    return out.reshape(BATCH, SEQ, D)


# speedup vs baseline: 1.0421x; 1.0421x over previous
import functools
import math

import jax
import jax.numpy as jnp
from jax import lax
from jax.experimental import pallas as pl
from jax.experimental.pallas import tpu as pltpu
from jax.experimental.pallas import tpu_sc as plsc

D = 128
SEQ = 200
BATCH = 4096
VOCAB = 4
NC, NS = 2, 16
NW = NC * NS
ROWS = BATCH * SEQ
RPW = ROWS // NW
CHUNK = 128
NCHUNK = RPW // CHUNK
GROUP = 2
NSTEP = NCHUNK // GROUP
NPAIR = NSTEP // 2
TROWS = SEQ * VOCAB


def _prep_body(table_ref, pe_ref, comb_ref):
    scale = jnp.float32(math.sqrt(float(D)))
    pe = pe_ref[...]
    for v in range(VOCAB):
        comb_ref[:, v, :] = pe + table_ref[v, :][None, :] * scale


def _sc_body(comb_hbm, tok_hbm, out_hbm, idx_v, buf_a, buf_b, comb_sh,
             gsem_a, gsem_b, ssem_a, ssem_b):
    cid = lax.axis_index("c")
    sid = lax.axis_index("s")
    wid = sid * NC + cid
    base = wid * RPW

    srows = 80

    @pl.when(sid < TROWS // srows)
    def _stage():
        off = pl.multiple_of(sid * srows, 8)
        pltpu.sync_copy(comb_hbm.at[pl.ds(off, srows)], buf_a.at[pl.ds(0, srows)])
        pltpu.sync_copy(buf_a.at[pl.ds(0, srows)], comb_sh.at[pl.ds(off, srows)])

    plsc.subcore_barrier()

    pltpu.sync_copy(tok_hbm.at[wid], idx_v)

    def convertpair(k):
        lanes = lax.iota(jnp.int32, 16)
        for r in range(2 * GROUP):
            j = 2 * GROUP * k + r
            for c in range(CHUNK // 16):
                off = c * 16
                tok16 = idx_v[j, pl.ds(off, 16)]
                pos = (lanes + (j * CHUNK + off)) % SEQ
                idx_v[j, pl.ds(off, 16)] = tok16 + VOCAB * pos

    def gather(s, buf, sem, issue):
        for g in range(GROUP):
            cp = pltpu.make_async_copy(
                comb_sh.at[idx_v.at[s * GROUP + g]],
                buf.at[pl.ds(g * CHUNK, CHUNK)],
                sem,
            )
            if issue:
                cp.start()
            else:
                cp.wait()

    def scatter(s, buf, sem, issue):
        cp = pltpu.make_async_copy(
            buf, out_hbm.at[pl.ds(base + s * GROUP * CHUNK, GROUP * CHUNK)], sem
        )
        if issue:
            cp.start()
        else:
            cp.wait()

    convertpair(0)
    gather(0, buf_a, gsem_a, True)

    def body(k, carry):
        s0 = 2 * k

        @pl.when(k > 0)
        def _():
            scatter(s0 - 1, buf_b, ssem_b, False)

        gather(s0 + 1, buf_b, gsem_b, True)

        @pl.when(k < NPAIR - 1)
        def _():
            convertpair(k + 1)

        gather(s0, buf_a, gsem_a, False)
        scatter(s0, buf_a, ssem_a, True)

        scatter(s0, buf_a, ssem_a, False)

        @pl.when(k < NPAIR - 1)
        def _():
            gather(s0 + 2, buf_a, gsem_a, True)

        gather(s0 + 1, buf_b, gsem_b, False)
        scatter(s0 + 1, buf_b, ssem_b, True)
        return carry

    lax.fori_loop(0, NPAIR, body, 0)
    scatter(NSTEP - 1, buf_b, ssem_b, False)


def kernel(token_indices, table, pe):
    comb = pl.pallas_call(
        _prep_body,
        out_shape=jax.ShapeDtypeStruct((SEQ, VOCAB, D), jnp.float32),
    )(table, pe[:SEQ])

    comb = comb.reshape(TROWS, D)
    tok3 = token_indices.reshape(NW, NCHUNK, CHUNK)

    sc = pl.kernel(
        _sc_body,
        out_type=jax.ShapeDtypeStruct((ROWS, D), jnp.float32),
        mesh=plsc.VectorSubcoreMesh(
            core_axis_name="c", subcore_axis_name="s", num_cores=NC, num_subcores=NS
        ),
        scratch_types=[
            pltpu.VMEM((NCHUNK, CHUNK), jnp.int32),
            pltpu.VMEM((GROUP * CHUNK, D), jnp.float32),
            pltpu.VMEM((GROUP * CHUNK, D), jnp.float32),
            pltpu.VMEM_SHARED((TROWS, D), jnp.float32),
            pltpu.SemaphoreType.DMA,
            pltpu.SemaphoreType.DMA,
            pltpu.SemaphoreType.DMA,
            pltpu.SemaphoreType.DMA,
        ],
    )
    out = sc(comb, tok3)
    return out.reshape(BATCH, SEQ, D)


# speedup vs baseline: 1.0550x; 1.0123x over previous
import functools
import math

import jax
import jax.numpy as jnp
from jax import lax
from jax.experimental import pallas as pl
from jax.experimental.pallas import tpu as pltpu
from jax.experimental.pallas import tpu_sc as plsc

D = 128
SEQ = 200
BATCH = 4096
VOCAB = 4
NC, NS = 2, 16
NW = NC * NS
ROWS = BATCH * SEQ
RPW = ROWS // NW
CHUNK = 128
NCHUNK = RPW // CHUNK
GROUP = 2
NSTEP = NCHUNK // GROUP
NTRI = NSTEP // 3
TROWS = SEQ * VOCAB


def _prep_body(table_ref, pe_ref, comb_ref):
    scale = jnp.float32(math.sqrt(float(D)))
    pe = pe_ref[...]
    for v in range(VOCAB):
        comb_ref[:, v, :] = pe + table_ref[v, :][None, :] * scale


def _sc_body(comb_hbm, tok_hbm, out_hbm, idx_v, buf_0, buf_1, buf_2, comb_sh,
             gsem_0, gsem_1, gsem_2, ssem_0, ssem_1, ssem_2):
    bufs = (buf_0, buf_1, buf_2)
    gsems = (gsem_0, gsem_1, gsem_2)
    ssems = (ssem_0, ssem_1, ssem_2)
    cid = lax.axis_index("c")
    sid = lax.axis_index("s")
    wid = sid * NC + cid
    base = wid * RPW

    srows = 80

    @pl.when(sid < TROWS // srows)
    def _stage():
        off = pl.multiple_of(sid * srows, 8)
        pltpu.sync_copy(comb_hbm.at[pl.ds(off, srows)], buf_0.at[pl.ds(0, srows)])
        pltpu.sync_copy(buf_0.at[pl.ds(0, srows)], comb_sh.at[pl.ds(off, srows)])

    plsc.subcore_barrier()

    pltpu.sync_copy(tok_hbm.at[wid], idx_v)

    def convertstep(s):
        lanes = lax.iota(jnp.int32, 16)
        for r in range(GROUP):
            row = GROUP * s + r
            for c in range(CHUNK // 16):
                off = c * 16
                tok16 = idx_v[row, pl.ds(off, 16)]
                pos = (lanes + (row * CHUNK + off)) % SEQ
                idx_v[row, pl.ds(off, 16)] = tok16 + VOCAB * pos

    def gather(s, buf, sem, issue):
        for g in range(GROUP):
            cp = pltpu.make_async_copy(
                comb_sh.at[idx_v.at[s * GROUP + g]],
                buf.at[pl.ds(g * CHUNK, CHUNK)],
                sem,
            )
            if issue:
                cp.start()
            else:
                cp.wait()

    def scatter(s, buf, sem, issue):
        cp = pltpu.make_async_copy(
            buf, out_hbm.at[pl.ds(base + s * GROUP * CHUNK, GROUP * CHUNK)], sem
        )
        if issue:
            cp.start()
        else:
            cp.wait()

    convertstep(0)
    convertstep(1)
    convertstep(2)
    gather(0, bufs[0], gsems[0], True)
    gather(1, bufs[1], gsems[1], True)

    def body(k, carry):
        for j in range(3):
            s = 3 * k + j
            bm1 = (j - 1) % 3
            bp2 = (j + 2) % 3

            if j == 0:
                @pl.when(k > 0)
                def _(s=s, bm1=bm1):
                    scatter(s - 1, bufs[bm1], ssems[bm1], False)
            else:
                scatter(s - 1, bufs[bm1], ssems[bm1], False)

            if j < 2:
                gather(s + 2, bufs[bp2], gsems[bp2], True)
            else:
                @pl.when(k < NTRI - 1)
                def _(s=s, bp2=bp2):
                    gather(s + 2, bufs[bp2], gsems[bp2], True)

            if j == 0:
                convertstep(s + 3)
            else:
                @pl.when(k < NTRI - 1)
                def _(s=s):
                    convertstep(s + 3)

            gather(s, bufs[j], gsems[j], False)
            scatter(s, bufs[j], ssems[j], True)
        return carry

    lax.fori_loop(0, NTRI, body, 0)

    scatter(NSTEP - 2, bufs[2], ssems[2], False)
    gather(NSTEP - 1, bufs[0], gsems[0], False)
    scatter(NSTEP - 1, bufs[0], ssems[0], True)
    scatter(NSTEP - 1, bufs[0], ssems[0], False)


def kernel(token_indices, table, pe):
    comb = pl.pallas_call(
        _prep_body,
        out_shape=jax.ShapeDtypeStruct((SEQ, VOCAB, D), jnp.float32),
    )(table, pe[:SEQ])

    comb = comb.reshape(TROWS, D)
    tok3 = token_indices.reshape(NW, NCHUNK, CHUNK)

    sc = pl.kernel(
        _sc_body,
        out_type=jax.ShapeDtypeStruct((ROWS, D), jnp.float32),
        mesh=plsc.VectorSubcoreMesh(
            core_axis_name="c", subcore_axis_name="s", num_cores=NC, num_subcores=NS
        ),
        scratch_types=[
            pltpu.VMEM((NCHUNK, CHUNK), jnp.int32),
            pltpu.VMEM((GROUP * CHUNK, D), jnp.float32),
            pltpu.VMEM((GROUP * CHUNK, D), jnp.float32),
            pltpu.VMEM((GROUP * CHUNK, D), jnp.float32),
            pltpu.VMEM_SHARED((TROWS, D), jnp.float32),
            pltpu.SemaphoreType.DMA,
            pltpu.SemaphoreType.DMA,
            pltpu.SemaphoreType.DMA,
            pltpu.SemaphoreType.DMA,
            pltpu.SemaphoreType.DMA,
            pltpu.SemaphoreType.DMA,
        ],
    )
    out = sc(comb, tok3)
    return out.reshape(BATCH, SEQ, D)
